# Initial kernel scaffold; baseline (speedup 1.0000x reference)
import jax
import jax.numpy as jnp
from jax.experimental import pallas as pl


def kernel(h, edge_index, edge_attr, W1, b1, W2, b2, W3, b3, W4, b4):
    raise NotImplementedError("write your pallas kernel here")


# speedup vs baseline: 2.1305x; 2.1305x over previous
import functools

import jax
import jax.numpy as jnp
from jax import lax
from jax.experimental import pallas as pl
from jax.experimental.pallas import tpu as pltpu
from jax.experimental.pallas import tpu_sc as plsc

N_NODES = 10000
N_EDGES = 320000
D_FEAT = 128
D_EDGE = 16
HIDDEN = 256
OUT_NF = 128

NW = 32
CH = 80
GCHUNKS = N_EDGES // (NW * CH)
SCHUNKS = N_EDGES // (16 * CH)
ACHUNKS = N_NODES // CH


def _silu(x):
    return x * jax.nn.sigmoid(x)


def _prep_body(h_ref, wa_ref, wb_ref, b1_ref, ha_ref, hb_ref):
    hblk = h_ref[...]
    ha_ref[...] = jnp.dot(hblk, wa_ref[...],
                          preferred_element_type=jnp.float32) + b1_ref[...]
    hb_ref[...] = jnp.dot(hblk, wb_ref[...],
                          preferred_element_type=jnp.float32)


def _prep(h, w1a, w1b, b1r):
    blk = 1000
    grid = N_NODES // blk
    return pl.pallas_call(
        _prep_body,
        grid=(grid,),
        in_specs=[
            pl.BlockSpec((blk, D_FEAT), lambda i: (i, 0)),
            pl.BlockSpec((D_FEAT, HIDDEN), lambda i: (0, 0)),
            pl.BlockSpec((D_FEAT, HIDDEN), lambda i: (0, 0)),
            pl.BlockSpec((1, HIDDEN), lambda i: (0, 0)),
        ],
        out_specs=[
            pl.BlockSpec((blk, HIDDEN), lambda i: (i, 0)),
            pl.BlockSpec((blk, HIDDEN), lambda i: (i, 0)),
        ],
        out_shape=[
            jax.ShapeDtypeStruct((N_NODES, HIDDEN), jnp.float32),
            jax.ShapeDtypeStruct((N_NODES, HIDDEN), jnp.float32),
        ],
    )(h, w1a, w1b, b1r)


def _gather(hA, hB, row3d, col3d):
    mesh = plsc.VectorSubcoreMesh(core_axis_name="c", subcore_axis_name="s")

    @functools.partial(
        pl.kernel,
        out_type=jax.ShapeDtypeStruct((N_EDGES, HIDDEN), jnp.float32),
        mesh=mesh,
        scratch_types=[
            pltpu.VMEM((GCHUNKS, CH), jnp.int32),
            pltpu.VMEM((GCHUNKS, CH), jnp.int32),
            pltpu.VMEM((CH, HIDDEN), jnp.float32),
            pltpu.VMEM((CH, HIDDEN), jnp.float32),
            pltpu.SemaphoreType.DMA,
            pltpu.SemaphoreType.DMA,
        ],
    )
    def k(hA_hbm, hB_hbm, row_hbm, col_hbm, out_hbm,
          idxA, idxB, bufA, bufB, semA, semB):
        wid = lax.axis_index("s") * 2 + lax.axis_index("c")
        pltpu.sync_copy(row_hbm.at[wid], idxA)
        pltpu.sync_copy(col_hbm.at[wid], idxB)

        def chunk(j, _):
            cpA = pltpu.async_copy(hA_hbm.at[idxA.at[j]], bufA, semA)
            cpB = pltpu.async_copy(hB_hbm.at[idxB.at[j]], bufB, semB)
            cpA.wait()
            cpB.wait()

            def addrow(r, _):
                for q in range(HIDDEN // 16):
                    ds = pl.ds(q * 16, 16)
                    bufA[r, ds] = bufA[r, ds] + bufB[r, ds]
                return 0

            lax.fori_loop(0, CH, addrow, 0)
            pltpu.sync_copy(
                bufA, out_hbm.at[pl.ds(wid * GCHUNKS * CH + j * CH, CH)])
            return 0

        lax.fori_loop(0, GCHUNKS, chunk, 0)

    return k(hA, hB, row3d, col3d)


def _edge_body(g_ref, attr_ref, w1c_ref, w2_ref, b2_ref, out_ref):
    pre = g_ref[...] + jnp.dot(attr_ref[...], w1c_ref[...],
                               preferred_element_type=jnp.float32)
    t = _silu(pre)
    m = jnp.dot(t, w2_ref[...], preferred_element_type=jnp.float32) + b2_ref[...]
    out_ref[...] = _silu(m)


def _edge(G, edge_attr, w1c, w2, b2r):
    blk = 512
    grid = N_EDGES // blk
    return pl.pallas_call(
        _edge_body,
        grid=(grid,),
        in_specs=[
            pl.BlockSpec((blk, HIDDEN), lambda i: (i, 0)),
            pl.BlockSpec((blk, D_EDGE), lambda i: (i, 0)),
            pl.BlockSpec((D_EDGE, HIDDEN), lambda i: (0, 0)),
            pl.BlockSpec((HIDDEN, HIDDEN), lambda i: (0, 0)),
            pl.BlockSpec((1, HIDDEN), lambda i: (0, 0)),
        ],
        out_specs=pl.BlockSpec((blk, HIDDEN), lambda i: (i, 0)),
        out_shape=jax.ShapeDtypeStruct((N_EDGES, HIDDEN), jnp.float32),
    )(G, edge_attr, w1c, w2, b2r)


def _scatter(mij, row3d16):
    mesh = plsc.VectorSubcoreMesh(core_axis_name="c", subcore_axis_name="s")

    @functools.partial(
        pl.kernel,
        out_type=jax.ShapeDtypeStruct((N_NODES, HIDDEN), jnp.float32),
        mesh=mesh,
        scratch_types=[
            pltpu.VMEM_SHARED((N_NODES, HIDDEN // 2), jnp.float32),
            pltpu.VMEM((SCHUNKS, CH), jnp.int32),
            pltpu.VMEM((CH, HIDDEN // 2), jnp.float32),
        ],
    )
    def k(mij_hbm, row_hbm, agg_hbm, acc, idx, buf):
        cid = lax.axis_index("c")
        sid = lax.axis_index("s")
        half = HIDDEN // 2

        def zrow(r, _):
            for q in range(half // 16):
                buf[r, pl.ds(q * 16, 16)] = jnp.zeros((16,), jnp.float32)
            return 0

        lax.fori_loop(0, CH, zrow, 0)
        for kk in range((ACHUNKS + 15) // 16):
            ch = sid + kk * 16

            @pl.when(ch < ACHUNKS)
            def _():
                pltpu.sync_copy(buf, acc.at[pl.ds(ch * CH, CH)])
        plsc.subcore_barrier()

        pltpu.sync_copy(row_hbm.at[sid], idx)

        def chunk(j, _):
            pltpu.sync_copy(
                mij_hbm.at[pl.ds(sid * SCHUNKS * CH + j * CH, CH),
                           pl.ds(cid * half, half)],
                buf)
            pltpu.sync_copy(buf, acc.at[idx.at[j]], add=True)
            return 0

        lax.fori_loop(0, SCHUNKS, chunk, 0)
        plsc.subcore_barrier()

        for kk in range((ACHUNKS + 15) // 16):
            ch = sid + kk * 16

            @pl.when(ch < ACHUNKS)
            def _():
                rs = pl.ds(ch * CH, CH)
                pltpu.sync_copy(acc.at[rs],
                                agg_hbm.at[rs, pl.ds(cid * half, half)])

    return k(mij, row3d16)


def _node_body(h_ref, agg_ref, w3a_ref, w3b_ref, b3_ref, w4_ref, b4_ref,
               out_ref):
    hblk = h_ref[...]
    hid = _silu(jnp.dot(hblk, w3a_ref[...], preferred_element_type=jnp.float32)
                + jnp.dot(agg_ref[...], w3b_ref[...],
                          preferred_element_type=jnp.float32)
                + b3_ref[...])
    out_ref[...] = hblk + jnp.dot(hid, w4_ref[...],
                                  preferred_element_type=jnp.float32) + b4_ref[...]


def _node(h, agg, w3a, w3b, b3r, w4, b4r):
    blk = 1000
    grid = N_NODES // blk
    return pl.pallas_call(
        _node_body,
        grid=(grid,),
        in_specs=[
            pl.BlockSpec((blk, D_FEAT), lambda i: (i, 0)),
            pl.BlockSpec((blk, HIDDEN), lambda i: (i, 0)),
            pl.BlockSpec((D_FEAT, HIDDEN), lambda i: (0, 0)),
            pl.BlockSpec((HIDDEN, HIDDEN), lambda i: (0, 0)),
            pl.BlockSpec((1, HIDDEN), lambda i: (0, 0)),
            pl.BlockSpec((HIDDEN, OUT_NF), lambda i: (0, 0)),
            pl.BlockSpec((1, OUT_NF), lambda i: (0, 0)),
        ],
        out_specs=pl.BlockSpec((blk, OUT_NF), lambda i: (i, 0)),
        out_shape=jax.ShapeDtypeStruct((N_NODES, OUT_NF), jnp.float32),
    )(h, agg, w3a, w3b, b3r, w4, b4r)


def kernel(h, edge_index, edge_attr, W1, b1, W2, b2, W3, b3, W4, b4):
    row = edge_index[0].astype(jnp.int32)
    col = edge_index[1].astype(jnp.int32)
    row3d = row.reshape(NW, GCHUNKS, CH)
    col3d = col.reshape(NW, GCHUNKS, CH)
    row3d16 = row.reshape(16, SCHUNKS, CH)

    w1a = W1[:D_FEAT]
    w1b = W1[D_FEAT:2 * D_FEAT]
    w1c = W1[2 * D_FEAT:]
    w3a = W3[:D_FEAT]
    w3b = W3[D_FEAT:]

    hA, hB = _prep(h, w1a, w1b, b1.reshape(1, HIDDEN))
    G = _gather(hA, hB, row3d, col3d)
    mij = _edge(G, edge_attr, w1c, W2, b2.reshape(1, HIDDEN))
    agg = _scatter(mij, row3d16)
    h_new = _node(h, agg, w3a, w3b, b3.reshape(1, HIDDEN), W4,
                  b4.reshape(1, OUT_NF))
    return (h_new, mij)


# speedup vs baseline: 2.7390x; 1.2856x over previous
import functools

import jax
import jax.numpy as jnp
from jax import lax
from jax.experimental import pallas as pl
from jax.experimental.pallas import tpu as pltpu
from jax.experimental.pallas import tpu_sc as plsc

N_NODES = 10000
N_EDGES = 320000
D_FEAT = 128
D_EDGE = 16
HIDDEN = 256
OUT_NF = 128

NW = 32
CH = 80
GCHUNKS = N_EDGES // (NW * CH)
SCHUNKS = N_EDGES // (16 * CH)
ACHUNKS = N_NODES // CH


def _silu(x):
    return x * jax.nn.sigmoid(x)


def _pack_halves(x32):
    m = x32.shape[1] // 2
    xr = x32.astype(jnp.bfloat16).astype(jnp.float32)
    return pltpu.pack_elementwise([xr[:, :m], xr[:, m:]],
                                  packed_dtype=jnp.bfloat16)


def _prep_body(h_ref, wa_ref, wb_ref, b1_ref, ha_ref, hb_ref):
    hblk = h_ref[...]
    ha_ref[...] = _pack_halves(
        jnp.dot(hblk, wa_ref[...], preferred_element_type=jnp.float32)
        + b1_ref[...])
    hb_ref[...] = _pack_halves(
        jnp.dot(hblk, wb_ref[...], preferred_element_type=jnp.float32))


def _prep(h, w1a, w1b, b1r):
    blk = 1000
    grid = N_NODES // blk
    return pl.pallas_call(
        _prep_body,
        grid=(grid,),
        in_specs=[
            pl.BlockSpec((blk, D_FEAT), lambda i: (i, 0)),
            pl.BlockSpec((D_FEAT, HIDDEN), lambda i: (0, 0)),
            pl.BlockSpec((D_FEAT, HIDDEN), lambda i: (0, 0)),
            pl.BlockSpec((1, HIDDEN), lambda i: (0, 0)),
        ],
        out_specs=[
            pl.BlockSpec((blk, HIDDEN // 2), lambda i: (i, 0)),
            pl.BlockSpec((blk, HIDDEN // 2), lambda i: (i, 0)),
        ],
        out_shape=[
            jax.ShapeDtypeStruct((N_NODES, HIDDEN // 2), jnp.int32),
            jax.ShapeDtypeStruct((N_NODES, HIDDEN // 2), jnp.int32),
        ],
    )(h, w1a, w1b, b1r)


def _gather(hA, hB, row3d, col3d):
    mesh = plsc.VectorSubcoreMesh(core_axis_name="c", subcore_axis_name="s")

    @functools.partial(
        pl.kernel,
        out_type=[
            jax.ShapeDtypeStruct((N_EDGES, HIDDEN // 2), jnp.int32),
            jax.ShapeDtypeStruct((N_EDGES, HIDDEN // 2), jnp.int32),
        ],
        mesh=mesh,
        scratch_types=[
            pltpu.VMEM((GCHUNKS, CH), jnp.int32),
            pltpu.VMEM((GCHUNKS, CH), jnp.int32),
            [pltpu.VMEM((CH, HIDDEN // 2), jnp.int32) for _ in range(2)],
            [pltpu.VMEM((CH, HIDDEN // 2), jnp.int32) for _ in range(2)],
            [pltpu.SemaphoreType.DMA for _ in range(2)],
            [pltpu.SemaphoreType.DMA for _ in range(2)],
            [pltpu.SemaphoreType.DMA for _ in range(2)],
            [pltpu.SemaphoreType.DMA for _ in range(2)],
        ],
    )
    def k(hA_hbm, hB_hbm, row_hbm, col_hbm, o1_hbm, o2_hbm,
          idxA, idxB, bA, bB, sA, sB, sO1, sO2):
        wid = lax.axis_index("s") * 2 + lax.axis_index("c")
        ebase = wid * GCHUNKS * CH
        pltpu.sync_copy(row_hbm.at[wid], idxA)
        pltpu.sync_copy(col_hbm.at[wid], idxB)

        def g_start(j, t):
            pltpu.async_copy(hA_hbm.at[idxA.at[j]], bA[t], sA[t])
            pltpu.async_copy(hB_hbm.at[idxB.at[j]], bB[t], sB[t])

        def g_wait(j, t):
            pltpu.make_async_copy(hA_hbm.at[idxA.at[j]], bA[t], sA[t]).wait()
            pltpu.make_async_copy(hB_hbm.at[idxB.at[j]], bB[t], sB[t]).wait()

        def w_start(j, t):
            pltpu.async_copy(bA[t], o1_hbm.at[pl.ds(ebase + j * CH, CH)],
                             sO1[t])
            pltpu.async_copy(bB[t], o2_hbm.at[pl.ds(ebase + j * CH, CH)],
                             sO2[t])

        def w_wait(j, t):
            pltpu.make_async_copy(
                bA[t], o1_hbm.at[pl.ds(ebase + j * CH, CH)], sO1[t]).wait()
            pltpu.make_async_copy(
                bB[t], o2_hbm.at[pl.ds(ebase + j * CH, CH)], sO2[t]).wait()

        g_start(0, 0)

        def pair(j2, _):
            for t in range(2):
                j = j2 * 2 + t
                g_wait(j, t)
                w_start(j, t)

                @pl.when(j > 0)
                def _():
                    w_wait(j - 1, 1 - t)

                @pl.when(j + 1 < GCHUNKS)
                def _():
                    g_start(j + 1, 1 - t)
            return 0

        lax.fori_loop(0, GCHUNKS // 2, pair, 0)
        jt = GCHUNKS - 1
        g_wait(jt, 0)
        w_start(jt, 0)
        w_wait(jt - 1, 1)
        w_wait(jt, 0)

    return k(hA, hB, row3d, col3d)


def _unpack2(gw):
    lo = pltpu.unpack_elementwise(gw, index=0, packed_dtype=jnp.bfloat16,
                                  unpacked_dtype=jnp.float32)
    hi = pltpu.unpack_elementwise(gw, index=1, packed_dtype=jnp.bfloat16,
                                  unpacked_dtype=jnp.float32)
    return lo, hi


def _edge_body(g1_ref, g2_ref, attr_ref, w1c_ref, w2_ref, b2_ref, out_ref):
    half = HIDDEN // 2
    a0, a1 = _unpack2(g1_ref[...])
    b0, b1 = _unpack2(g2_ref[...])
    attrc = jnp.dot(attr_ref[...], w1c_ref[...],
                    preferred_element_type=jnp.float32)
    t0 = _silu(a0 + b0 + attrc[:, :half]).astype(jnp.bfloat16)
    t1 = _silu(a1 + b1 + attrc[:, half:]).astype(jnp.bfloat16)
    w2 = w2_ref[...]
    m = (jnp.dot(t0, w2[:half], preferred_element_type=jnp.float32)
         + jnp.dot(t1, w2[half:], preferred_element_type=jnp.float32)
         + b2_ref[...])
    out_ref[...] = _silu(m)


def _edge(G1, G2, edge_attr, w1c, w2, b2r):
    blk = 512
    grid = N_EDGES // blk
    return pl.pallas_call(
        _edge_body,
        grid=(grid,),
        in_specs=[
            pl.BlockSpec((blk, HIDDEN // 2), lambda i: (i, 0)),
            pl.BlockSpec((blk, HIDDEN // 2), lambda i: (i, 0)),
            pl.BlockSpec((blk, D_EDGE), lambda i: (i, 0)),
            pl.BlockSpec((D_EDGE, HIDDEN), lambda i: (0, 0)),
            pl.BlockSpec((HIDDEN, HIDDEN), lambda i: (0, 0)),
            pl.BlockSpec((1, HIDDEN), lambda i: (0, 0)),
        ],
        out_specs=pl.BlockSpec((blk, HIDDEN), lambda i: (i, 0)),
        out_shape=jax.ShapeDtypeStruct((N_EDGES, HIDDEN), jnp.float32),
    )(G1, G2, edge_attr, w1c, w2, b2r)


def _scatter(mij, row3d):
    mesh = plsc.VectorSubcoreMesh(core_axis_name="c", subcore_axis_name="s")

    @functools.partial(
        pl.kernel,
        out_type=jax.ShapeDtypeStruct((N_NODES, HIDDEN), jnp.float32),
        mesh=mesh,
        scratch_types=[
            pltpu.VMEM_SHARED((N_NODES, HIDDEN // 2), jnp.float32),
            pltpu.VMEM((GCHUNKS, CH), jnp.int32),
            [pltpu.VMEM((CH, HIDDEN // 2), jnp.float32) for _ in range(2)],
            [pltpu.SemaphoreType.DMA for _ in range(2)],
        ],
    )
    def k(mij_hbm, row_hbm, agg_hbm, acc, idx, buf, sem):
        cid = lax.axis_index("c")
        sid = lax.axis_index("s")
        half = HIDDEN // 2

        def zrow(r, _):
            for q in range(half // 16):
                buf[0][r, pl.ds(q * 16, 16)] = jnp.zeros((16,), jnp.float32)
            return 0

        lax.fori_loop(0, CH, zrow, 0)
        for kk in range((ACHUNKS + 15) // 16):
            ch = sid + kk * 16

            @pl.when(ch < ACHUNKS)
            def _():
                pltpu.sync_copy(buf[0], acc.at[pl.ds(ch * CH, CH)])
        plsc.subcore_barrier()

        for p in range(2):
            pltpu.sync_copy(row_hbm.at[sid * 2 + p], idx)

            def m_src(j):
                return mij_hbm.at[
                    pl.ds((sid * 2 + p) * GCHUNKS * CH + j * CH, CH),
                    pl.ds(cid * half, half)]

            pltpu.async_copy(m_src(0), buf[0], sem[0])

            def chunk2(j2, _):
                for t in range(2):
                    j = j2 * 2 + t
                    pltpu.make_async_copy(m_src(j), buf[t], sem[t]).wait()

                    @pl.when(j + 1 < GCHUNKS)
                    def _():
                        pltpu.async_copy(m_src(j + 1), buf[1 - t],
                                         sem[1 - t])
                    pltpu.sync_copy(buf[t], acc.at[idx.at[j]], add=True)
                return 0

            lax.fori_loop(0, GCHUNKS // 2, chunk2, 0)
            jt = GCHUNKS - 1
            pltpu.make_async_copy(m_src(jt), buf[0], sem[0]).wait()
            pltpu.sync_copy(buf[0], acc.at[idx.at[jt]], add=True)
        plsc.subcore_barrier()

        for kk in range((ACHUNKS + 15) // 16):
            ch = sid + kk * 16

            @pl.when(ch < ACHUNKS)
            def _():
                rs = pl.ds(ch * CH, CH)
                pltpu.sync_copy(acc.at[rs],
                                agg_hbm.at[rs, pl.ds(cid * half, half)])

    return k(mij, row3d)


def _node_body(h_ref, agg_ref, w3a_ref, w3b_ref, b3_ref, w4_ref, b4_ref,
               out_ref):
    hblk = h_ref[...]
    hid = _silu(jnp.dot(hblk, w3a_ref[...], preferred_element_type=jnp.float32)
                + jnp.dot(agg_ref[...], w3b_ref[...],
                          preferred_element_type=jnp.float32)
                + b3_ref[...])
    out_ref[...] = hblk + jnp.dot(hid, w4_ref[...],
                                  preferred_element_type=jnp.float32) + b4_ref[...]


def _node(h, agg, w3a, w3b, b3r, w4, b4r):
    blk = 1000
    grid = N_NODES // blk
    return pl.pallas_call(
        _node_body,
        grid=(grid,),
        in_specs=[
            pl.BlockSpec((blk, D_FEAT), lambda i: (i, 0)),
            pl.BlockSpec((blk, HIDDEN), lambda i: (i, 0)),
            pl.BlockSpec((D_FEAT, HIDDEN), lambda i: (0, 0)),
            pl.BlockSpec((HIDDEN, HIDDEN), lambda i: (0, 0)),
            pl.BlockSpec((1, HIDDEN), lambda i: (0, 0)),
            pl.BlockSpec((HIDDEN, OUT_NF), lambda i: (0, 0)),
            pl.BlockSpec((1, OUT_NF), lambda i: (0, 0)),
        ],
        out_specs=pl.BlockSpec((blk, OUT_NF), lambda i: (i, 0)),
        out_shape=jax.ShapeDtypeStruct((N_NODES, OUT_NF), jnp.float32),
    )(h, agg, w3a, w3b, b3r, w4, b4r)


def kernel(h, edge_index, edge_attr, W1, b1, W2, b2, W3, b3, W4, b4):
    row = edge_index[0].astype(jnp.int32)
    col = edge_index[1].astype(jnp.int32)
    row3d = row.reshape(NW, GCHUNKS, CH)
    col3d = col.reshape(NW, GCHUNKS, CH)

    w1a = W1[:D_FEAT]
    w1b = W1[D_FEAT:2 * D_FEAT]
    w1c = W1[2 * D_FEAT:]
    w3a = W3[:D_FEAT]
    w3b = W3[D_FEAT:]

    hA, hB = _prep(h, w1a, w1b, b1.reshape(1, HIDDEN))
    G1, G2 = _gather(hA, hB, row3d, col3d)
    mij = _edge(G1, G2, edge_attr, w1c, W2.astype(jnp.bfloat16),
                b2.reshape(1, HIDDEN))
    agg = _scatter(mij, row3d)
    h_new = _node(h, agg, w3a, w3b, b3.reshape(1, HIDDEN), W4,
                  b4.reshape(1, OUT_NF))
    return (h_new, mij)


# speedup vs baseline: 3.0146x; 1.1006x over previous
import functools

import jax
import jax.numpy as jnp
from jax import lax
from jax.experimental import pallas as pl
from jax.experimental.pallas import tpu as pltpu
from jax.experimental.pallas import tpu_sc as plsc

N_NODES = 10000
N_EDGES = 320000
D_FEAT = 128
D_EDGE = 16
HIDDEN = 256
OUT_NF = 128

NW = 32
CH = 80
GCHUNKS = N_EDGES // (NW * CH)
SCHUNKS = N_EDGES // (16 * CH)
ACHUNKS = N_NODES // CH


def _silu(x):
    return x * jax.nn.sigmoid(x)


def _pack_halves(x32):
    m = x32.shape[1] // 2
    xr = x32.astype(jnp.bfloat16).astype(jnp.float32)
    return pltpu.pack_elementwise([xr[:, :m], xr[:, m:]],
                                  packed_dtype=jnp.bfloat16)


def _prep_body(h_ref, wa_ref, wb_ref, b1_ref, ha_ref, hb_ref):
    hblk = h_ref[...]
    ha_ref[...] = _pack_halves(
        jnp.dot(hblk, wa_ref[...], preferred_element_type=jnp.float32)
        + b1_ref[...])
    hb_ref[...] = _pack_halves(
        jnp.dot(hblk, wb_ref[...], preferred_element_type=jnp.float32))


def _prep(h, w1a, w1b, b1r):
    blk = 1000
    grid = N_NODES // blk
    return pl.pallas_call(
        _prep_body,
        grid=(grid,),
        in_specs=[
            pl.BlockSpec((blk, D_FEAT), lambda i: (i, 0)),
            pl.BlockSpec((D_FEAT, HIDDEN), lambda i: (0, 0)),
            pl.BlockSpec((D_FEAT, HIDDEN), lambda i: (0, 0)),
            pl.BlockSpec((1, HIDDEN), lambda i: (0, 0)),
        ],
        out_specs=[
            pl.BlockSpec((blk, HIDDEN // 2), lambda i: (i, 0)),
            pl.BlockSpec((blk, HIDDEN // 2), lambda i: (i, 0)),
        ],
        out_shape=[
            jax.ShapeDtypeStruct((N_NODES, HIDDEN // 2), jnp.int32),
            jax.ShapeDtypeStruct((N_NODES, HIDDEN // 2), jnp.int32),
        ],
    )(h, w1a, w1b, b1r)


def _gather(hA, hB, row3d, col3d):
    mesh = plsc.VectorSubcoreMesh(core_axis_name="c", subcore_axis_name="s")

    @functools.partial(
        pl.kernel,
        out_type=[
            jax.ShapeDtypeStruct((N_EDGES, HIDDEN // 2), jnp.int32),
            jax.ShapeDtypeStruct((N_EDGES, HIDDEN // 2), jnp.int32),
        ],
        mesh=mesh,
        scratch_types=[
            pltpu.VMEM((GCHUNKS, CH), jnp.int32),
            pltpu.VMEM((GCHUNKS, CH), jnp.int32),
            [pltpu.VMEM((CH, HIDDEN // 2), jnp.int32) for _ in range(2)],
            [pltpu.VMEM((CH, HIDDEN // 2), jnp.int32) for _ in range(2)],
            [pltpu.SemaphoreType.DMA for _ in range(2)],
            [pltpu.SemaphoreType.DMA for _ in range(2)],
            [pltpu.SemaphoreType.DMA for _ in range(2)],
            [pltpu.SemaphoreType.DMA for _ in range(2)],
        ],
    )
    def k(hA_hbm, hB_hbm, row_hbm, col_hbm, o1_hbm, o2_hbm,
          idxA, idxB, bA, bB, sA, sB, sO1, sO2):
        wid = lax.axis_index("s") * 2 + lax.axis_index("c")
        ebase = wid * GCHUNKS * CH
        pltpu.sync_copy(row_hbm.at[wid], idxA)
        pltpu.sync_copy(col_hbm.at[wid], idxB)

        def g_start(j, t):
            pltpu.async_copy(hA_hbm.at[idxA.at[j]], bA[t], sA[t])
            pltpu.async_copy(hB_hbm.at[idxB.at[j]], bB[t], sB[t])

        def g_wait(j, t):
            pltpu.make_async_copy(hA_hbm.at[idxA.at[j]], bA[t], sA[t]).wait()
            pltpu.make_async_copy(hB_hbm.at[idxB.at[j]], bB[t], sB[t]).wait()

        def w_start(j, t):
            pltpu.async_copy(bA[t], o1_hbm.at[pl.ds(ebase + j * CH, CH)],
                             sO1[t])
            pltpu.async_copy(bB[t], o2_hbm.at[pl.ds(ebase + j * CH, CH)],
                             sO2[t])

        def w_wait(j, t):
            pltpu.make_async_copy(
                bA[t], o1_hbm.at[pl.ds(ebase + j * CH, CH)], sO1[t]).wait()
            pltpu.make_async_copy(
                bB[t], o2_hbm.at[pl.ds(ebase + j * CH, CH)], sO2[t]).wait()

        g_start(0, 0)

        def pair(j2, _):
            for t in range(2):
                j = j2 * 2 + t
                g_wait(j, t)
                w_start(j, t)

                @pl.when(j > 0)
                def _():
                    w_wait(j - 1, 1 - t)

                @pl.when(j + 1 < GCHUNKS)
                def _():
                    g_start(j + 1, 1 - t)
            return 0

        lax.fori_loop(0, GCHUNKS // 2, pair, 0)
        jt = GCHUNKS - 1
        g_wait(jt, 0)
        w_start(jt, 0)
        w_wait(jt - 1, 1)
        w_wait(jt, 0)

    return k(hA, hB, row3d, col3d)


def _unpack2(gw):
    lo = pltpu.unpack_elementwise(gw, index=0, packed_dtype=jnp.bfloat16,
                                  unpacked_dtype=jnp.float32)
    hi = pltpu.unpack_elementwise(gw, index=1, packed_dtype=jnp.bfloat16,
                                  unpacked_dtype=jnp.float32)
    return lo, hi


def _edge_body(g1_ref, g2_ref, attr_ref, w1c_ref, w2_ref, b2_ref, out_ref):
    half = HIDDEN // 2
    a0, a1 = _unpack2(g1_ref[...])
    b0, b1 = _unpack2(g2_ref[...])
    attrc = jnp.dot(attr_ref[...].astype(jnp.bfloat16), w1c_ref[...],
                    preferred_element_type=jnp.float32)
    t0 = _silu(a0 + b0 + attrc[:, :half]).astype(jnp.bfloat16)
    t1 = _silu(a1 + b1 + attrc[:, half:]).astype(jnp.bfloat16)
    w2 = w2_ref[...]
    m = (jnp.dot(t0, w2[:half], preferred_element_type=jnp.float32)
         + jnp.dot(t1, w2[half:], preferred_element_type=jnp.float32)
         + b2_ref[...])
    out_ref[...] = _silu(m)


def _edge(G1, G2, edge_attr, w1c, w2, b2r):
    blk = 800
    grid = N_EDGES // blk
    return pl.pallas_call(
        _edge_body,
        grid=(grid,),
        in_specs=[
            pl.BlockSpec((blk, HIDDEN // 2), lambda i: (i, 0)),
            pl.BlockSpec((blk, HIDDEN // 2), lambda i: (i, 0)),
            pl.BlockSpec((blk, D_EDGE), lambda i: (i, 0)),
            pl.BlockSpec((D_EDGE, HIDDEN), lambda i: (0, 0)),
            pl.BlockSpec((HIDDEN, HIDDEN), lambda i: (0, 0)),
            pl.BlockSpec((1, HIDDEN), lambda i: (0, 0)),
        ],
        out_specs=pl.BlockSpec((blk, HIDDEN), lambda i: (i, 0)),
        out_shape=jax.ShapeDtypeStruct((N_EDGES, HIDDEN), jnp.float32),
    )(G1, G2, edge_attr, w1c, w2, b2r)


def _scatter(mij, row3d):
    mesh = plsc.VectorSubcoreMesh(core_axis_name="c", subcore_axis_name="s")

    @functools.partial(
        pl.kernel,
        out_type=jax.ShapeDtypeStruct((N_NODES, HIDDEN), jnp.float32),
        mesh=mesh,
        scratch_types=[
            pltpu.VMEM_SHARED((N_NODES, HIDDEN // 2), jnp.float32),
            pltpu.VMEM((GCHUNKS, CH), jnp.int32),
            [pltpu.VMEM((CH, HIDDEN // 2), jnp.float32) for _ in range(2)],
            [pltpu.SemaphoreType.DMA for _ in range(2)],
        ],
    )
    def k(mij_hbm, row_hbm, agg_hbm, acc, idx, buf, sem):
        cid = lax.axis_index("c")
        sid = lax.axis_index("s")
        half = HIDDEN // 2

        def zrow(r, _):
            for q in range(half // 16):
                buf[0][r, pl.ds(q * 16, 16)] = jnp.zeros((16,), jnp.float32)
            return 0

        lax.fori_loop(0, CH, zrow, 0)
        for kk in range((ACHUNKS + 15) // 16):
            ch = sid + kk * 16

            @pl.when(ch < ACHUNKS)
            def _():
                pltpu.sync_copy(buf[0], acc.at[pl.ds(ch * CH, CH)])
        plsc.subcore_barrier()

        for p in range(2):
            pltpu.sync_copy(row_hbm.at[sid * 2 + p], idx)

            def m_src(j):
                return mij_hbm.at[
                    pl.ds((sid * 2 + p) * GCHUNKS * CH + j * CH, CH),
                    pl.ds(cid * half, half)]

            pltpu.async_copy(m_src(0), buf[0], sem[0])

            def chunk2(j2, _):
                for t in range(2):
                    j = j2 * 2 + t
                    pltpu.make_async_copy(m_src(j), buf[t], sem[t]).wait()

                    @pl.when(j + 1 < GCHUNKS)
                    def _():
                        pltpu.async_copy(m_src(j + 1), buf[1 - t],
                                         sem[1 - t])
                    pltpu.sync_copy(buf[t], acc.at[idx.at[j]], add=True)
                return 0

            lax.fori_loop(0, GCHUNKS // 2, chunk2, 0)
            jt = GCHUNKS - 1
            pltpu.make_async_copy(m_src(jt), buf[0], sem[0]).wait()
            pltpu.sync_copy(buf[0], acc.at[idx.at[jt]], add=True)
        plsc.subcore_barrier()

        for kk in range((ACHUNKS + 15) // 16):
            ch = sid + kk * 16

            @pl.when(ch < ACHUNKS)
            def _():
                rs = pl.ds(ch * CH, CH)
                pltpu.sync_copy(acc.at[rs],
                                agg_hbm.at[rs, pl.ds(cid * half, half)])

    return k(mij, row3d)


def _node_body(h_ref, agg_ref, w3a_ref, w3b_ref, b3_ref, w4_ref, b4_ref,
               out_ref):
    hblk = h_ref[...]
    hid = _silu(jnp.dot(hblk, w3a_ref[...], preferred_element_type=jnp.float32)
                + jnp.dot(agg_ref[...], w3b_ref[...],
                          preferred_element_type=jnp.float32)
                + b3_ref[...])
    out_ref[...] = hblk + jnp.dot(hid, w4_ref[...],
                                  preferred_element_type=jnp.float32) + b4_ref[...]


def _node(h, agg, w3a, w3b, b3r, w4, b4r):
    blk = 1000
    grid = N_NODES // blk
    return pl.pallas_call(
        _node_body,
        grid=(grid,),
        in_specs=[
            pl.BlockSpec((blk, D_FEAT), lambda i: (i, 0)),
            pl.BlockSpec((blk, HIDDEN), lambda i: (i, 0)),
            pl.BlockSpec((D_FEAT, HIDDEN), lambda i: (0, 0)),
            pl.BlockSpec((HIDDEN, HIDDEN), lambda i: (0, 0)),
            pl.BlockSpec((1, HIDDEN), lambda i: (0, 0)),
            pl.BlockSpec((HIDDEN, OUT_NF), lambda i: (0, 0)),
            pl.BlockSpec((1, OUT_NF), lambda i: (0, 0)),
        ],
        out_specs=pl.BlockSpec((blk, OUT_NF), lambda i: (i, 0)),
        out_shape=jax.ShapeDtypeStruct((N_NODES, OUT_NF), jnp.float32),
    )(h, agg, w3a, w3b, b3r, w4, b4r)


def kernel(h, edge_index, edge_attr, W1, b1, W2, b2, W3, b3, W4, b4):
    row = edge_index[0].astype(jnp.int32)
    col = edge_index[1].astype(jnp.int32)
    row3d = row.reshape(NW, GCHUNKS, CH)
    col3d = col.reshape(NW, GCHUNKS, CH)

    w1a = W1[:D_FEAT]
    w1b = W1[D_FEAT:2 * D_FEAT]
    w1c = W1[2 * D_FEAT:]
    w3a = W3[:D_FEAT]
    w3b = W3[D_FEAT:]

    hA, hB = _prep(h, w1a, w1b, b1.reshape(1, HIDDEN))
    G1, G2 = _gather(hA, hB, row3d, col3d)
    mij = _edge(G1, G2, edge_attr, w1c.astype(jnp.bfloat16),
                W2.astype(jnp.bfloat16), b2.reshape(1, HIDDEN))
    agg = _scatter(mij, row3d)
    h_new = _node(h, agg, w3a, w3b, b3.reshape(1, HIDDEN), W4,
                  b4.reshape(1, OUT_NF))
    return (h_new, mij)


# speedup vs baseline: 3.5915x; 1.1914x over previous
import functools

import jax
import jax.numpy as jnp
from jax import lax
from jax.experimental import pallas as pl
from jax.experimental.pallas import tpu as pltpu
from jax.experimental.pallas import tpu_sc as plsc

N_NODES = 10000
N_EDGES = 320000
D_FEAT = 128
D_EDGE = 16
HIDDEN = 256
OUT_NF = 128

NW = 32
CH = 80
GCHUNKS = N_EDGES // (NW * CH)
ACHUNKS = N_NODES // CH
NSTRIP = 5
ESTRIP = N_EDGES // NSTRIP
GSCH = ESTRIP // (NW * CH)
SCHUNKS = ESTRIP // (16 * CH)


def _silu(x):
    return x * jax.nn.sigmoid(x)


def _pack_halves(x32):
    m = x32.shape[1] // 2
    xr = x32.astype(jnp.bfloat16).astype(jnp.float32)
    return pltpu.pack_elementwise([xr[:, :m], xr[:, m:]],
                                  packed_dtype=jnp.bfloat16)


def _prep_body(h_ref, wa_ref, wb_ref, b1_ref, ha_ref, hb_ref):
    hblk = h_ref[...]
    ha_ref[...] = _pack_halves(
        jnp.dot(hblk, wa_ref[...], preferred_element_type=jnp.float32)
        + b1_ref[...])
    hb_ref[...] = _pack_halves(
        jnp.dot(hblk, wb_ref[...], preferred_element_type=jnp.float32))


def _prep(h, w1a, w1b, b1r):
    blk = 1000
    grid = N_NODES // blk
    return pl.pallas_call(
        _prep_body,
        grid=(grid,),
        in_specs=[
            pl.BlockSpec((blk, D_FEAT), lambda i: (i, 0)),
            pl.BlockSpec((D_FEAT, HIDDEN), lambda i: (0, 0)),
            pl.BlockSpec((D_FEAT, HIDDEN), lambda i: (0, 0)),
            pl.BlockSpec((1, HIDDEN), lambda i: (0, 0)),
        ],
        out_specs=[
            pl.BlockSpec((blk, HIDDEN // 2), lambda i: (i, 0)),
            pl.BlockSpec((blk, HIDDEN // 2), lambda i: (i, 0)),
        ],
        out_shape=[
            jax.ShapeDtypeStruct((N_NODES, HIDDEN // 2), jnp.int32),
            jax.ShapeDtypeStruct((N_NODES, HIDDEN // 2), jnp.int32),
        ],
    )(h, w1a, w1b, b1r)


def _gather_strip(hA, hB, row4d, col4d, s):
    mesh = plsc.VectorSubcoreMesh(core_axis_name="c", subcore_axis_name="s")

    @functools.partial(
        pl.kernel,
        out_type=[
            jax.ShapeDtypeStruct((ESTRIP, HIDDEN // 2), jnp.int32),
            jax.ShapeDtypeStruct((ESTRIP, HIDDEN // 2), jnp.int32),
        ],
        mesh=mesh,
        scratch_types=[
            pltpu.VMEM((GSCH, CH), jnp.int32),
            pltpu.VMEM((GSCH, CH), jnp.int32),
            [pltpu.VMEM((CH, HIDDEN // 2), jnp.int32) for _ in range(2)],
            [pltpu.VMEM((CH, HIDDEN // 2), jnp.int32) for _ in range(2)],
            [pltpu.SemaphoreType.DMA for _ in range(2)],
            [pltpu.SemaphoreType.DMA for _ in range(2)],
            [pltpu.SemaphoreType.DMA for _ in range(2)],
            [pltpu.SemaphoreType.DMA for _ in range(2)],
        ],
    )
    def k(hA_hbm, hB_hbm, row_hbm, col_hbm, o1_hbm, o2_hbm,
          idxA, idxB, bA, bB, sA, sB, sO1, sO2):
        wid = lax.axis_index("s") * 2 + lax.axis_index("c")
        ebase = wid * GSCH * CH
        pltpu.sync_copy(row_hbm.at[s * NW + wid], idxA)
        pltpu.sync_copy(col_hbm.at[s * NW + wid], idxB)

        def g_start(j, t):
            pltpu.async_copy(hA_hbm.at[idxA.at[j]], bA[t], sA[t])
            pltpu.async_copy(hB_hbm.at[idxB.at[j]], bB[t], sB[t])

        def g_wait(j, t):
            pltpu.make_async_copy(hA_hbm.at[idxA.at[j]], bA[t], sA[t]).wait()
            pltpu.make_async_copy(hB_hbm.at[idxB.at[j]], bB[t], sB[t]).wait()

        def w_start(j, t):
            pltpu.async_copy(bA[t], o1_hbm.at[pl.ds(ebase + j * CH, CH)],
                             sO1[t])
            pltpu.async_copy(bB[t], o2_hbm.at[pl.ds(ebase + j * CH, CH)],
                             sO2[t])

        def w_wait(j, t):
            pltpu.make_async_copy(
                bA[t], o1_hbm.at[pl.ds(ebase + j * CH, CH)], sO1[t]).wait()
            pltpu.make_async_copy(
                bB[t], o2_hbm.at[pl.ds(ebase + j * CH, CH)], sO2[t]).wait()

        g_start(0, 0)

        def pair(j2, _):
            for t in range(2):
                j = j2 * 2 + t
                g_wait(j, t)
                w_start(j, t)

                @pl.when(j > 0)
                def _():
                    w_wait(j - 1, 1 - t)

                @pl.when(j + 1 < GSCH)
                def _():
                    g_start(j + 1, 1 - t)
            return 0

        lax.fori_loop(0, GSCH // 2, pair, 0)
        jt = GSCH - 1
        g_wait(jt, 0)
        w_start(jt, 0)
        w_wait(jt - 1, 1)
        w_wait(jt, 0)

    return k(hA, hB, row4d, col4d)


def _unpack2(gw):
    lo = pltpu.unpack_elementwise(gw, index=0, packed_dtype=jnp.bfloat16,
                                  unpacked_dtype=jnp.float32)
    hi = pltpu.unpack_elementwise(gw, index=1, packed_dtype=jnp.bfloat16,
                                  unpacked_dtype=jnp.float32)
    return lo, hi


def _edge_compute(g1_ref, g2_ref, attr_ref, w1c_ref, w2_ref, b2_ref):
    half = HIDDEN // 2
    a0, a1 = _unpack2(g1_ref[...])
    b0, b1 = _unpack2(g2_ref[...])
    attrc = jnp.dot(attr_ref[...].astype(jnp.bfloat16), w1c_ref[...],
                    preferred_element_type=jnp.float32)
    t0 = _silu(a0 + b0 + attrc[:, :half]).astype(jnp.bfloat16)
    t1 = _silu(a1 + b1 + attrc[:, half:]).astype(jnp.bfloat16)
    w2 = w2_ref[...]
    m = (jnp.dot(t0, w2[:half], preferred_element_type=jnp.float32)
         + jnp.dot(t1, w2[half:], preferred_element_type=jnp.float32)
         + b2_ref[...])
    return _silu(m)


def _edge_body(g1_ref, g2_ref, attr_ref, w1c_ref, w2_ref, b2_ref,
               strip_ref, full_ref):
    v = _edge_compute(g1_ref, g2_ref, attr_ref, w1c_ref, w2_ref, b2_ref)
    strip_ref[...] = v
    full_ref[...] = v


def _edge_body_aliased(prev_ref, g1_ref, g2_ref, attr_ref, w1c_ref, w2_ref,
                       b2_ref, strip_ref, full_ref):
    del prev_ref
    v = _edge_compute(g1_ref, g2_ref, attr_ref, w1c_ref, w2_ref, b2_ref)
    strip_ref[...] = v
    full_ref[...] = v


def _edge_strip(mij_prev, G1s, G2s, edge_attr, w1c, w2, b2r, s):
    blk = 800
    grid = ESTRIP // blk
    base = s * (ESTRIP // blk)
    body = _edge_body if mij_prev is None else _edge_body_aliased
    in_specs = [
        pl.BlockSpec((blk, HIDDEN // 2), lambda i: (i, 0)),
        pl.BlockSpec((blk, HIDDEN // 2), lambda i: (i, 0)),
        pl.BlockSpec((blk, D_EDGE), lambda i: (base + i, 0)),
        pl.BlockSpec((D_EDGE, HIDDEN), lambda i: (0, 0)),
        pl.BlockSpec((HIDDEN, HIDDEN), lambda i: (0, 0)),
        pl.BlockSpec((1, HIDDEN), lambda i: (0, 0)),
    ]
    args = (G1s, G2s, edge_attr, w1c, w2, b2r)
    aliases = {}
    if mij_prev is not None:
        in_specs = [pl.BlockSpec(memory_space=pltpu.MemorySpace.HBM)] + in_specs
        args = (mij_prev,) + args
        aliases = {0: 1}
    return pl.pallas_call(
        body,
        grid=(grid,),
        in_specs=in_specs,
        out_specs=[
            pl.BlockSpec((blk, HIDDEN), lambda i: (i, 0)),
            pl.BlockSpec((blk, HIDDEN), lambda i: (base + i, 0)),
        ],
        out_shape=[
            jax.ShapeDtypeStruct((ESTRIP, HIDDEN), jnp.float32),
            jax.ShapeDtypeStruct((N_EDGES, HIDDEN), jnp.float32),
        ],
        input_output_aliases=aliases,
    )(*args)


def _scatter_strip(mij, row3d16, s):
    mesh = plsc.VectorSubcoreMesh(core_axis_name="c", subcore_axis_name="s")

    @functools.partial(
        pl.kernel,
        out_type=jax.ShapeDtypeStruct((N_NODES, HIDDEN), jnp.float32),
        mesh=mesh,
        scratch_types=[
            pltpu.VMEM_SHARED((N_NODES, HIDDEN // 2), jnp.float32),
            pltpu.VMEM((SCHUNKS, CH), jnp.int32),
            [pltpu.VMEM((CH, HIDDEN // 2), jnp.float32) for _ in range(2)],
            [pltpu.SemaphoreType.DMA for _ in range(2)],
        ],
    )
    def k(mij_hbm, row_hbm, agg_hbm, acc, idx, buf, sem):
        cid = lax.axis_index("c")
        sid = lax.axis_index("s")
        half = HIDDEN // 2

        def zrow(r, _):
            for q in range(half // 16):
                buf[0][r, pl.ds(q * 16, 16)] = jnp.zeros((16,), jnp.float32)
            return 0

        lax.fori_loop(0, CH, zrow, 0)
        for kk in range((ACHUNKS + 15) // 16):
            ch = sid + kk * 16

            @pl.when(ch < ACHUNKS)
            def _():
                pltpu.sync_copy(buf[0], acc.at[pl.ds(ch * CH, CH)])
        plsc.subcore_barrier()

        pltpu.sync_copy(row_hbm.at[s * 16 + sid], idx)
        ebase = sid * SCHUNKS * CH

        def m_src(j):
            return mij_hbm.at[pl.ds(ebase + j * CH, CH),
                              pl.ds(cid * half, half)]

        pltpu.async_copy(m_src(0), buf[0], sem[0])

        def chunk2(j2, _):
            for t in range(2):
                j = j2 * 2 + t
                pltpu.make_async_copy(m_src(j), buf[t], sem[t]).wait()

                @pl.when(j + 1 < SCHUNKS)
                def _():
                    pltpu.async_copy(m_src(j + 1), buf[1 - t], sem[1 - t])
                pltpu.sync_copy(buf[t], acc.at[idx.at[j]], add=True)
            return 0

        lax.fori_loop(0, SCHUNKS // 2, chunk2, 0)
        plsc.subcore_barrier()

        for kk in range((ACHUNKS + 15) // 16):
            ch = sid + kk * 16

            @pl.when(ch < ACHUNKS)
            def _():
                rs = pl.ds(ch * CH, CH)
                pltpu.sync_copy(acc.at[rs],
                                agg_hbm.at[rs, pl.ds(cid * half, half)])

    return k(mij, row3d16)


def _node_body(h_ref, *rest):
    agg_refs = rest[:NSTRIP]
    w3a_ref, w3b_ref, b3_ref, w4_ref, b4_ref, out_ref = rest[NSTRIP:]
    hblk = h_ref[...]
    agg = agg_refs[0][...]
    for a in agg_refs[1:]:
        agg = agg + a[...]
    hid = _silu(jnp.dot(hblk, w3a_ref[...], preferred_element_type=jnp.float32)
                + jnp.dot(agg, w3b_ref[...],
                          preferred_element_type=jnp.float32)
                + b3_ref[...])
    out_ref[...] = hblk + jnp.dot(hid, w4_ref[...],
                                  preferred_element_type=jnp.float32) + b4_ref[...]


def _node(h, aggs, w3a, w3b, b3r, w4, b4r):
    blk = 1000
    grid = N_NODES // blk
    return pl.pallas_call(
        _node_body,
        grid=(grid,),
        in_specs=[
            pl.BlockSpec((blk, D_FEAT), lambda i: (i, 0)),
        ] + [
            pl.BlockSpec((blk, HIDDEN), lambda i: (i, 0))
            for _ in range(NSTRIP)
        ] + [
            pl.BlockSpec((D_FEAT, HIDDEN), lambda i: (0, 0)),
            pl.BlockSpec((HIDDEN, HIDDEN), lambda i: (0, 0)),
            pl.BlockSpec((1, HIDDEN), lambda i: (0, 0)),
            pl.BlockSpec((HIDDEN, OUT_NF), lambda i: (0, 0)),
            pl.BlockSpec((1, OUT_NF), lambda i: (0, 0)),
        ],
        out_specs=pl.BlockSpec((blk, OUT_NF), lambda i: (i, 0)),
        out_shape=jax.ShapeDtypeStruct((N_NODES, OUT_NF), jnp.float32),
    )(h, *aggs, w3a, w3b, b3r, w4, b4r)


def kernel(h, edge_index, edge_attr, W1, b1, W2, b2, W3, b3, W4, b4):
    row = edge_index[0].astype(jnp.int32)
    col = edge_index[1].astype(jnp.int32)
    row4d = row.reshape(NSTRIP * NW, GSCH, CH)
    col4d = col.reshape(NSTRIP * NW, GSCH, CH)
    row3d16 = row.reshape(NSTRIP * 16, SCHUNKS, CH)

    w1a = W1[:D_FEAT]
    w1b = W1[D_FEAT:2 * D_FEAT]
    w1c = W1[2 * D_FEAT:]
    w3a = W3[:D_FEAT]
    w3b = W3[D_FEAT:]

    hA, hB = _prep(h, w1a, w1b, b1.reshape(1, HIDDEN))

    w1cb = w1c.astype(jnp.bfloat16)
    w2b = W2.astype(jnp.bfloat16)
    b2r = b2.reshape(1, HIDDEN)
    gs = [_gather_strip(hA, hB, row4d, col4d, s) for s in range(NSTRIP)]
    mij = None
    aggs = []
    for s in range(NSTRIP):
        mij_s, mij = _edge_strip(mij, gs[s][0], gs[s][1], edge_attr,
                                 w1cb, w2b, b2r, s)
        aggs.append(_scatter_strip(mij_s, row3d16, s))
    h_new = _node(h, aggs, w3a, w3b, b3.reshape(1, HIDDEN), W4,
                  b4.reshape(1, OUT_NF))
    return (h_new, mij)


# speedup vs baseline: 3.9388x; 1.0967x over previous
import functools

import jax
import jax.numpy as jnp
from jax import lax
from jax.experimental import pallas as pl
from jax.experimental.pallas import tpu as pltpu
from jax.experimental.pallas import tpu_sc as plsc

N_NODES = 10000
N_EDGES = 320000
D_FEAT = 128
D_EDGE = 16
HIDDEN = 256
OUT_NF = 128

NW = 32
CH = 80
GCHUNKS = N_EDGES // (NW * CH)
ACHUNKS = N_NODES // CH
NSTRIP = 5
ESTRIP = N_EDGES // NSTRIP
GSCH2 = ESTRIP // (16 * CH)
SCHUNKS = ESTRIP // (16 * CH)


def _silu(x):
    return x * jax.nn.sigmoid(x)


def _pack_halves(x32):
    m = x32.shape[1] // 2
    xr = x32.astype(jnp.bfloat16).astype(jnp.float32)
    return pltpu.pack_elementwise([xr[:, :m], xr[:, m:]],
                                  packed_dtype=jnp.bfloat16)


def _prep_body(h_ref, wa_ref, wb_ref, b1_ref, ha_ref, hb_ref):
    hblk = h_ref[...]
    ha_ref[...] = _pack_halves(
        jnp.dot(hblk, wa_ref[...], preferred_element_type=jnp.float32)
        + b1_ref[...])
    hb_ref[...] = _pack_halves(
        jnp.dot(hblk, wb_ref[...], preferred_element_type=jnp.float32))


def _prep(h, w1a, w1b, b1r):
    blk = 1000
    grid = N_NODES // blk
    return pl.pallas_call(
        _prep_body,
        grid=(grid,),
        in_specs=[
            pl.BlockSpec((blk, D_FEAT), lambda i: (i, 0)),
            pl.BlockSpec((D_FEAT, HIDDEN), lambda i: (0, 0)),
            pl.BlockSpec((D_FEAT, HIDDEN), lambda i: (0, 0)),
            pl.BlockSpec((1, HIDDEN), lambda i: (0, 0)),
        ],
        out_specs=[
            pl.BlockSpec((blk, HIDDEN // 2), lambda i: (i, 0)),
            pl.BlockSpec((blk, HIDDEN // 2), lambda i: (i, 0)),
        ],
        out_shape=[
            jax.ShapeDtypeStruct((N_NODES, HIDDEN // 2), jnp.int32),
            jax.ShapeDtypeStruct((N_NODES, HIDDEN // 2), jnp.int32),
        ],
    )(h, w1a, w1b, b1r)


def _gather_strip(hAB, rc4d, s):
    mesh = plsc.VectorSubcoreMesh(core_axis_name="c", subcore_axis_name="s")

    @functools.partial(
        pl.kernel,
        out_type=jax.ShapeDtypeStruct((2, ESTRIP, HIDDEN // 2), jnp.int32),
        mesh=mesh,
        scratch_types=[
            pltpu.VMEM_SHARED((N_NODES, HIDDEN // 2), jnp.int32),
            pltpu.VMEM((GSCH2, CH), jnp.int32),
            [pltpu.VMEM((CH, HIDDEN // 2), jnp.int32) for _ in range(2)],
            [pltpu.SemaphoreType.DMA for _ in range(2)],
            [pltpu.SemaphoreType.DMA for _ in range(2)],
        ],
    )
    def k(hAB_hbm, rc_hbm, out_hbm, tbl, idx, bA, sG, sO):
        cid = lax.axis_index("c")
        sid = lax.axis_index("s")
        for kk in range((ACHUNKS + 15) // 16):
            ch = sid + kk * 16

            @pl.when(ch < ACHUNKS)
            def _():
                rs = pl.ds(ch * CH, CH)
                pltpu.sync_copy(hAB_hbm.at[cid].at[rs], tbl.at[rs])
        pltpu.sync_copy(rc_hbm.at[cid * (NSTRIP * 16) + s * 16 + sid], idx)
        plsc.subcore_barrier()
        ebase = sid * GSCH2 * CH

        def g_start(j, t):
            pltpu.async_copy(tbl.at[idx.at[j]], bA[t], sG[t])

        def g_wait(j, t):
            pltpu.make_async_copy(tbl.at[idx.at[j]], bA[t], sG[t]).wait()

        def o_dst(j):
            return out_hbm.at[cid].at[pl.ds(ebase + j * CH, CH)]

        def w_wait(j, t):
            pltpu.make_async_copy(bA[t], o_dst(j), sO[t]).wait()

        g_start(0, 0)

        def pair(j2, _):
            for t in range(2):
                j = j2 * 2 + t
                g_wait(j, t)
                pltpu.async_copy(bA[t], o_dst(j), sO[t])

                @pl.when(j > 0)
                def _():
                    w_wait(j - 1, 1 - t)

                @pl.when(j + 1 < GSCH2)
                def _():
                    g_start(j + 1, 1 - t)
            return 0

        lax.fori_loop(0, GSCH2 // 2, pair, 0)
        w_wait(GSCH2 - 1, 1)

    return k(hAB, rc4d)


def _unpack2(gw):
    lo = pltpu.unpack_elementwise(gw, index=0, packed_dtype=jnp.bfloat16,
                                  unpacked_dtype=jnp.float32)
    hi = pltpu.unpack_elementwise(gw, index=1, packed_dtype=jnp.bfloat16,
                                  unpacked_dtype=jnp.float32)
    return lo, hi


def _edge_compute(g1_ref, g2_ref, attr_ref, w1c_ref, w2_ref, b2_ref):
    half = HIDDEN // 2
    a0, a1 = _unpack2(jnp.squeeze(g1_ref[...], axis=0))
    b0, b1 = _unpack2(jnp.squeeze(g2_ref[...], axis=0))
    attrc = jnp.dot(attr_ref[...].astype(jnp.bfloat16), w1c_ref[...],
                    preferred_element_type=jnp.float32)
    t0 = _silu(a0 + b0 + attrc[:, :half]).astype(jnp.bfloat16)
    t1 = _silu(a1 + b1 + attrc[:, half:]).astype(jnp.bfloat16)
    w2 = w2_ref[...]
    m = (jnp.dot(t0, w2[:half], preferred_element_type=jnp.float32)
         + jnp.dot(t1, w2[half:], preferred_element_type=jnp.float32)
         + b2_ref[...])
    return _silu(m)


def _edge_body(g1_ref, g2_ref, attr_ref, w1c_ref, w2_ref, b2_ref,
               strip_ref, full_ref):
    v = _edge_compute(g1_ref, g2_ref, attr_ref, w1c_ref, w2_ref, b2_ref)
    strip_ref[...] = v
    full_ref[...] = v


def _edge_body_aliased(prev_ref, g1_ref, g2_ref, attr_ref, w1c_ref, w2_ref,
                       b2_ref, strip_ref, full_ref):
    del prev_ref
    v = _edge_compute(g1_ref, g2_ref, attr_ref, w1c_ref, w2_ref, b2_ref)
    strip_ref[...] = v
    full_ref[...] = v


def _edge_strip(mij_prev, G1s, G2s, edge_attr, w1c, w2, b2r, s):
    blk = 800
    grid = ESTRIP // blk
    base = s * (ESTRIP // blk)
    body = _edge_body if mij_prev is None else _edge_body_aliased
    in_specs = [
        pl.BlockSpec((1, blk, HIDDEN // 2), lambda i: (0, i, 0)),
        pl.BlockSpec((1, blk, HIDDEN // 2), lambda i: (1, i, 0)),
        pl.BlockSpec((blk, D_EDGE), lambda i: (base + i, 0)),
        pl.BlockSpec((D_EDGE, HIDDEN), lambda i: (0, 0)),
        pl.BlockSpec((HIDDEN, HIDDEN), lambda i: (0, 0)),
        pl.BlockSpec((1, HIDDEN), lambda i: (0, 0)),
    ]
    args = (G1s, G2s, edge_attr, w1c, w2, b2r)
    aliases = {}
    if mij_prev is not None:
        in_specs = [pl.BlockSpec(memory_space=pltpu.MemorySpace.HBM)] + in_specs
        args = (mij_prev,) + args
        aliases = {0: 1}
    return pl.pallas_call(
        body,
        grid=(grid,),
        in_specs=in_specs,
        out_specs=[
            pl.BlockSpec((blk, HIDDEN), lambda i: (i, 0)),
            pl.BlockSpec((blk, HIDDEN), lambda i: (base + i, 0)),
        ],
        out_shape=[
            jax.ShapeDtypeStruct((ESTRIP, HIDDEN), jnp.float32),
            jax.ShapeDtypeStruct((N_EDGES, HIDDEN), jnp.float32),
        ],
        input_output_aliases=aliases,
    )(*args)


def _scatter_strip(mij, row3d16, s):
    mesh = plsc.VectorSubcoreMesh(core_axis_name="c", subcore_axis_name="s")

    @functools.partial(
        pl.kernel,
        out_type=jax.ShapeDtypeStruct((N_NODES, HIDDEN), jnp.float32),
        mesh=mesh,
        scratch_types=[
            pltpu.VMEM_SHARED((N_NODES, HIDDEN // 2), jnp.float32),
            pltpu.VMEM((SCHUNKS, CH), jnp.int32),
            [pltpu.VMEM((CH, HIDDEN // 2), jnp.float32) for _ in range(2)],
            [pltpu.SemaphoreType.DMA for _ in range(2)],
        ],
    )
    def k(mij_hbm, row_hbm, agg_hbm, acc, idx, buf, sem):
        cid = lax.axis_index("c")
        sid = lax.axis_index("s")
        half = HIDDEN // 2

        def zrow(r, _):
            for q in range(half // 16):
                buf[0][r, pl.ds(q * 16, 16)] = jnp.zeros((16,), jnp.float32)
            return 0

        lax.fori_loop(0, CH, zrow, 0)
        for kk in range((ACHUNKS + 15) // 16):
            ch = sid + kk * 16

            @pl.when(ch < ACHUNKS)
            def _():
                pltpu.sync_copy(buf[0], acc.at[pl.ds(ch * CH, CH)])
        plsc.subcore_barrier()

        pltpu.sync_copy(row_hbm.at[s * 16 + sid], idx)
        ebase = sid * SCHUNKS * CH

        def m_src(j):
            return mij_hbm.at[pl.ds(ebase + j * CH, CH),
                              pl.ds(cid * half, half)]

        pltpu.async_copy(m_src(0), buf[0], sem[0])

        def chunk2(j2, _):
            for t in range(2):
                j = j2 * 2 + t
                pltpu.make_async_copy(m_src(j), buf[t], sem[t]).wait()

                @pl.when(j + 1 < SCHUNKS)
                def _():
                    pltpu.async_copy(m_src(j + 1), buf[1 - t], sem[1 - t])
                pltpu.sync_copy(buf[t], acc.at[idx.at[j]], add=True)
            return 0

        lax.fori_loop(0, SCHUNKS // 2, chunk2, 0)
        plsc.subcore_barrier()

        for kk in range((ACHUNKS + 15) // 16):
            ch = sid + kk * 16

            @pl.when(ch < ACHUNKS)
            def _():
                rs = pl.ds(ch * CH, CH)
                pltpu.sync_copy(acc.at[rs],
                                agg_hbm.at[rs, pl.ds(cid * half, half)])

    return k(mij, row3d16)


def _node_body(h_ref, *rest):
    agg_refs = rest[:NSTRIP]
    w3a_ref, w3b_ref, b3_ref, w4_ref, b4_ref, out_ref = rest[NSTRIP:]
    hblk = h_ref[...]
    agg = agg_refs[0][...]
    for a in agg_refs[1:]:
        agg = agg + a[...]
    hid = _silu(jnp.dot(hblk, w3a_ref[...], preferred_element_type=jnp.float32)
                + jnp.dot(agg, w3b_ref[...],
                          preferred_element_type=jnp.float32)
                + b3_ref[...])
    out_ref[...] = hblk + jnp.dot(hid, w4_ref[...],
                                  preferred_element_type=jnp.float32) + b4_ref[...]


def _node(h, aggs, w3a, w3b, b3r, w4, b4r):
    blk = 1000
    grid = N_NODES // blk
    return pl.pallas_call(
        _node_body,
        grid=(grid,),
        in_specs=[
            pl.BlockSpec((blk, D_FEAT), lambda i: (i, 0)),
        ] + [
            pl.BlockSpec((blk, HIDDEN), lambda i: (i, 0))
            for _ in range(NSTRIP)
        ] + [
            pl.BlockSpec((D_FEAT, HIDDEN), lambda i: (0, 0)),
            pl.BlockSpec((HIDDEN, HIDDEN), lambda i: (0, 0)),
            pl.BlockSpec((1, HIDDEN), lambda i: (0, 0)),
            pl.BlockSpec((HIDDEN, OUT_NF), lambda i: (0, 0)),
            pl.BlockSpec((1, OUT_NF), lambda i: (0, 0)),
        ],
        out_specs=pl.BlockSpec((blk, OUT_NF), lambda i: (i, 0)),
        out_shape=jax.ShapeDtypeStruct((N_NODES, OUT_NF), jnp.float32),
    )(h, *aggs, w3a, w3b, b3r, w4, b4r)


def kernel(h, edge_index, edge_attr, W1, b1, W2, b2, W3, b3, W4, b4):
    ei32 = edge_index.astype(jnp.int32)
    row = ei32[0]
    rc4d = ei32.reshape(2 * NSTRIP * 16, GSCH2, CH)
    row3d16 = row.reshape(NSTRIP * 16, SCHUNKS, CH)

    w1a = W1[:D_FEAT]
    w1b = W1[D_FEAT:2 * D_FEAT]
    w1c = W1[2 * D_FEAT:]
    w3a = W3[:D_FEAT]
    w3b = W3[D_FEAT:]

    hA, hB = _prep(h, w1a, w1b, b1.reshape(1, HIDDEN))

    w1cb = w1c.astype(jnp.bfloat16)
    w2b = W2.astype(jnp.bfloat16)
    b2r = b2.reshape(1, HIDDEN)
    hAB = jnp.stack([hA, hB])
    gs = [_gather_strip(hAB, rc4d, s) for s in range(NSTRIP)]
    mij = None
    aggs = []
    for s in range(NSTRIP):
        mij_s, mij = _edge_strip(mij, gs[s], gs[s], edge_attr,
                                 w1cb, w2b, b2r, s)
        aggs.append(_scatter_strip(mij_s, row3d16, s))
    h_new = _node(h, aggs, w3a, w3b, b3.reshape(1, HIDDEN), W4,
                  b4.reshape(1, OUT_NF))
    return (h_new, mij)
